# initial kernel scaffold (unmeasured)
import jax
import jax.numpy as jnp
from jax import lax
from jax.experimental import pallas as pl
from jax.experimental.pallas import tpu as pltpu

N_DEV = 16
B = 2
SQ = 256
SKV_LOC = 256
SKV = SKV_LOC * N_DEV
H_LOC = 4
H = 64
DH = 64
DM = 512
BLK = 64
RS_ROWS = SQ // N_DEV


def kernel(x, Wq, K_ext, V_ext, Wo):
    def body(
        x_ref, wq_ref, k_ref, v_ref, wo_ref, out_ref,
        kv_stack, kv_gath, po_ref, rs_buf, red_ref, ag_buf,
        kv_send, kv_recv, rs_send, rs_recv, ag_send, ag_recv,
    ):
        my = lax.axis_index("i")

        kv_stack[0] = jnp.transpose(
            k_ref[...].astype(jnp.bfloat16), (0, 2, 1, 3)
        )
        kv_stack[1] = jnp.transpose(
            v_ref[...].astype(jnp.bfloat16), (0, 2, 1, 3)
        )

        kv_sends = []
        for k in range(1, N_DEV):
            peer = lax.rem(my + k, N_DEV)
            rdma = pltpu.make_async_remote_copy(
                src_ref=kv_stack.at[:, :, pl.ds(H_LOC * peer, H_LOC)],
                dst_ref=kv_gath.at[my],
                send_sem=kv_send.at[peer],
                recv_sem=kv_recv.at[my],
                device_id=(peer,),
                device_id_type=pl.DeviceIdType.MESH,
            )
            rdma.start()
            kv_sends.append(rdma)

        own = kv_stack[:, :, pl.ds(H_LOC * my, H_LOC)]
        kv_gath[pl.ds(my, 1)] = own[None]

        for k in range(1, N_DEV):
            src = lax.rem(my - k + N_DEV, N_DEV)
            pltpu.make_async_remote_copy(
                src_ref=kv_stack.at[:, :, pl.ds(0, H_LOC)],
                dst_ref=kv_gath.at[src],
                send_sem=kv_send.at[src],
                recv_sem=kv_recv.at[src],
                device_id=(src,),
                device_id_type=pl.DeviceIdType.MESH,
            ).wait_recv()
        for rdma in kv_sends:
            rdma.wait_send()

        xb = x_ref[...].astype(jnp.bfloat16).reshape(B * SQ, DM)
        wqb = wq_ref[...].astype(jnp.bfloat16)
        q = jax.lax.dot_general(
            xb, wqb, (((1,), (0,)), ((), ())),
            preferred_element_type=jnp.float32,
        ).reshape(B, SQ, H_LOC, DH)

        qb = lax.broadcasted_iota(jnp.int32, (SQ, SKV), 0) // BLK
        kb = lax.broadcasted_iota(jnp.int32, (SQ, SKV), 1) // BLK
        mask = (qb == kb) | (kb == 0) | (lax.rem(qb + kb, 3) == 0)

        wob = wo_ref[...].astype(jnp.bfloat16)
        po = []
        for b in range(B):
            acc = jnp.zeros((SQ, DM), jnp.float32)
            for h in range(H_LOC):
                kf = kv_gath[:, 0, b, h].reshape(SKV, DH)
                vf = kv_gath[:, 1, b, h].reshape(SKV, DH)
                qh = q[b, :, h, :].astype(jnp.bfloat16)
                s = jax.lax.dot_general(
                    qh, kf, (((1,), (1,)), ((), ())),
                    preferred_element_type=jnp.float32,
                )
                s = jnp.where(mask, s * 0.125, -1e9)
                m = jnp.max(s, axis=-1, keepdims=True)
                w = jnp.exp(s - m)
                w = w / jnp.sum(w, axis=-1, keepdims=True)
                ctx = jax.lax.dot_general(
                    w.astype(jnp.bfloat16), vf, (((1,), (0,)), ((), ())),
                    preferred_element_type=jnp.float32,
                )
                acc = acc + jax.lax.dot_general(
                    ctx.astype(jnp.bfloat16),
                    wob[h * DH:(h + 1) * DH, :],
                    (((1,), (0,)), ((), ())),
                    preferred_element_type=jnp.float32,
                )
            po.append(acc)
        po_val = jnp.stack(po)
        po_ref[...] = po_val

        rs_sends = []
        for k in range(1, N_DEV):
            peer = lax.rem(my + k, N_DEV)
            rdma = pltpu.make_async_remote_copy(
                src_ref=po_ref.at[:, pl.ds(RS_ROWS * peer, RS_ROWS), :],
                dst_ref=rs_buf.at[my],
                send_sem=rs_send.at[peer],
                recv_sem=rs_recv.at[my],
                device_id=(peer,),
                device_id_type=pl.DeviceIdType.MESH,
            )
            rdma.start()
            rs_sends.append(rdma)
        own_chunk = lax.dynamic_slice(
            po_val, (0, RS_ROWS * my, 0), (B, RS_ROWS, DM)
        )
        rs_buf[pl.ds(my, 1)] = own_chunk[None]
        for k in range(1, N_DEV):
            src = lax.rem(my - k + N_DEV, N_DEV)
            pltpu.make_async_remote_copy(
                src_ref=po_ref.at[:, pl.ds(0, RS_ROWS), :],
                dst_ref=rs_buf.at[src],
                send_sem=rs_send.at[src],
                recv_sem=rs_recv.at[src],
                device_id=(src,),
                device_id_type=pl.DeviceIdType.MESH,
            ).wait_recv()
        for rdma in rs_sends:
            rdma.wait_send()

        red = jnp.sum(rs_buf[...], axis=0)
        red_ref[...] = red

        ag_sends = []
        for k in range(1, N_DEV):
            peer = lax.rem(my + k, N_DEV)
            rdma = pltpu.make_async_remote_copy(
                src_ref=red_ref,
                dst_ref=ag_buf.at[my],
                send_sem=ag_send.at[peer],
                recv_sem=ag_recv.at[my],
                device_id=(peer,),
                device_id_type=pl.DeviceIdType.MESH,
            )
            rdma.start()
            ag_sends.append(rdma)
        ag_buf[pl.ds(my, 1)] = red[None]
        for k in range(1, N_DEV):
            src = lax.rem(my - k + N_DEV, N_DEV)
            pltpu.make_async_remote_copy(
                src_ref=red_ref,
                dst_ref=ag_buf.at[src],
                send_sem=ag_send.at[src],
                recv_sem=ag_recv.at[src],
                device_id=(src,),
                device_id_type=pl.DeviceIdType.MESH,
            ).wait_recv()
        for rdma in ag_sends:
            rdma.wait_send()

        out_ref[...] = jnp.transpose(
            ag_buf[...], (1, 0, 2, 3)
        ).reshape(B, SQ, DM)

    vmem = pl.BlockSpec(memory_space=pltpu.VMEM)
    return pl.pallas_call(
        body,
        out_shape=jax.ShapeDtypeStruct((B, SQ, DM), jnp.float32),
        in_specs=[vmem] * 5,
        out_specs=vmem,
        scratch_shapes=[
            pltpu.VMEM((2, B, H, SKV_LOC, DH), jnp.bfloat16),
            pltpu.VMEM((N_DEV, 2, B, H_LOC, SKV_LOC, DH), jnp.bfloat16),
            pltpu.VMEM((B, SQ, DM), jnp.float32),
            pltpu.VMEM((N_DEV, B, RS_ROWS, DM), jnp.float32),
            pltpu.VMEM((B, RS_ROWS, DM), jnp.float32),
            pltpu.VMEM((N_DEV, B, RS_ROWS, DM), jnp.float32),
            pltpu.SemaphoreType.DMA((N_DEV,)),
            pltpu.SemaphoreType.DMA((N_DEV,)),
            pltpu.SemaphoreType.DMA((N_DEV,)),
            pltpu.SemaphoreType.DMA((N_DEV,)),
            pltpu.SemaphoreType.DMA((N_DEV,)),
            pltpu.SemaphoreType.DMA((N_DEV,)),
        ],
        compiler_params=pltpu.CompilerParams(collective_id=0),
    )(x, Wq, K_ext, V_ext, Wo)


# baseline (device time: 215385 ns/iter reference)
import jax
import jax.numpy as jnp
from jax import lax
from jax.experimental import pallas as pl
from jax.experimental.pallas import tpu as pltpu

N_DEV = 16
B = 2
SQ = 256
SKV_LOC = 256
SKV = SKV_LOC * N_DEV
H_LOC = 4
H = 64
DH = 64
DM = 512
BLK = 64
RS_ROWS = SQ // N_DEV
HD_LOC = H_LOC * DH


def kernel(x, Wq, K_ext, V_ext, Wo):
    def body(
        x_ref, wq_ref, k_ref, v_ref, wo_ref, out_ref,
        staging, kv_stack, kv_gath, po_ref, rs_buf, red_ref, ag_buf,
        copy_sem,
        kv_send, kv_recv, rs_send, rs_recv, ag_send, ag_recv,
    ):
        my = lax.axis_index("i")

        for kvi, hbm in ((0, k_ref), (1, v_ref)):
            for b in range(B):
                cp = pltpu.make_async_copy(hbm.at[b], staging, copy_sem)
                cp.start()
                cp.wait()
                kv_stack[kvi, b] = (
                    staging[...].astype(jnp.bfloat16).reshape(SKV_LOC, H * DH)
                )

        kv_sends = []
        for k in range(1, N_DEV):
            peer = lax.rem(my + k, N_DEV)
            rdma = pltpu.make_async_remote_copy(
                src_ref=kv_stack.at[:, :, :, pl.ds(HD_LOC * peer, HD_LOC)],
                dst_ref=kv_gath.at[my],
                send_sem=kv_send.at[peer],
                recv_sem=kv_recv.at[my],
                device_id=(peer,),
                device_id_type=pl.DeviceIdType.MESH,
            )
            rdma.start()
            kv_sends.append(rdma)

        own = kv_stack[:, :, :, pl.ds(HD_LOC * my, HD_LOC)]
        kv_gath[pl.ds(my, 1)] = own[None]

        for k in range(1, N_DEV):
            src = lax.rem(my - k + N_DEV, N_DEV)
            pltpu.make_async_remote_copy(
                src_ref=kv_stack.at[:, :, :, pl.ds(0, HD_LOC)],
                dst_ref=kv_gath.at[src],
                send_sem=kv_send.at[src],
                recv_sem=kv_recv.at[src],
                device_id=(src,),
                device_id_type=pl.DeviceIdType.MESH,
            ).wait_recv()
        for rdma in kv_sends:
            rdma.wait_send()

        xb = x_ref[...].astype(jnp.bfloat16).reshape(B * SQ, DM)
        wqb = wq_ref[...].astype(jnp.bfloat16)
        q = jax.lax.dot_general(
            xb, wqb, (((1,), (0,)), ((), ())),
            preferred_element_type=jnp.float32,
        )

        qb = lax.broadcasted_iota(jnp.int32, (SQ, SKV), 0) // BLK
        kb = lax.broadcasted_iota(jnp.int32, (SQ, SKV), 1) // BLK
        mask = (qb == kb) | (kb == 0) | (lax.rem(qb + kb, 3) == 0)

        wob = wo_ref[...].astype(jnp.bfloat16)
        po = []
        for b in range(B):
            acc = jnp.zeros((SQ, DM), jnp.float32)
            for h in range(H_LOC):
                kf = kv_gath[:, 0, b, :, pl.ds(DH * h, DH)].reshape(SKV, DH)
                vf = kv_gath[:, 1, b, :, pl.ds(DH * h, DH)].reshape(SKV, DH)
                qh = q[
                    b * SQ:(b + 1) * SQ, DH * h:DH * (h + 1)
                ].astype(jnp.bfloat16)
                s = jax.lax.dot_general(
                    qh, kf, (((1,), (1,)), ((), ())),
                    preferred_element_type=jnp.float32,
                )
                s = jnp.where(mask, s * 0.125, -1e9)
                m = jnp.max(s, axis=-1, keepdims=True)
                w = jnp.exp(s - m)
                w = w / jnp.sum(w, axis=-1, keepdims=True)
                ctx = jax.lax.dot_general(
                    w.astype(jnp.bfloat16), vf, (((1,), (0,)), ((), ())),
                    preferred_element_type=jnp.float32,
                )
                acc = acc + jax.lax.dot_general(
                    ctx.astype(jnp.bfloat16),
                    wob[h * DH:(h + 1) * DH, :],
                    (((1,), (0,)), ((), ())),
                    preferred_element_type=jnp.float32,
                )
            po.append(acc)
        po_ref[...] = jnp.stack(po)

        rs_sends = []
        for k in range(1, N_DEV):
            peer = lax.rem(my + k, N_DEV)
            rdma = pltpu.make_async_remote_copy(
                src_ref=po_ref.at[:, pl.ds(RS_ROWS * peer, RS_ROWS), :],
                dst_ref=rs_buf.at[my],
                send_sem=rs_send.at[peer],
                recv_sem=rs_recv.at[my],
                device_id=(peer,),
                device_id_type=pl.DeviceIdType.MESH,
            )
            rdma.start()
            rs_sends.append(rdma)
        own_chunk = po_ref[:, pl.ds(RS_ROWS * my, RS_ROWS), :]
        rs_buf[pl.ds(my, 1)] = own_chunk[None]
        for k in range(1, N_DEV):
            src = lax.rem(my - k + N_DEV, N_DEV)
            pltpu.make_async_remote_copy(
                src_ref=po_ref.at[:, pl.ds(0, RS_ROWS), :],
                dst_ref=rs_buf.at[src],
                send_sem=rs_send.at[src],
                recv_sem=rs_recv.at[src],
                device_id=(src,),
                device_id_type=pl.DeviceIdType.MESH,
            ).wait_recv()
        for rdma in rs_sends:
            rdma.wait_send()

        red = jnp.sum(rs_buf[...], axis=0)
        red_ref[...] = red

        ag_sends = []
        for k in range(1, N_DEV):
            peer = lax.rem(my + k, N_DEV)
            rdma = pltpu.make_async_remote_copy(
                src_ref=red_ref,
                dst_ref=ag_buf.at[my],
                send_sem=ag_send.at[peer],
                recv_sem=ag_recv.at[my],
                device_id=(peer,),
                device_id_type=pl.DeviceIdType.MESH,
            )
            rdma.start()
            ag_sends.append(rdma)
        ag_buf[pl.ds(my, 1)] = red[None]
        for k in range(1, N_DEV):
            src = lax.rem(my - k + N_DEV, N_DEV)
            pltpu.make_async_remote_copy(
                src_ref=red_ref,
                dst_ref=ag_buf.at[src],
                send_sem=ag_send.at[src],
                recv_sem=ag_recv.at[src],
                device_id=(src,),
                device_id_type=pl.DeviceIdType.MESH,
            ).wait_recv()
        for rdma in ag_sends:
            rdma.wait_send()

        out_ref[...] = jnp.transpose(
            ag_buf[...], (1, 0, 2, 3)
        ).reshape(B, SQ, DM)

    vmem = pl.BlockSpec(memory_space=pltpu.VMEM)
    hbm = pl.BlockSpec(memory_space=pltpu.MemorySpace.HBM)
    return pl.pallas_call(
        body,
        out_shape=jax.ShapeDtypeStruct((B, SQ, DM), jnp.float32),
        in_specs=[vmem, vmem, hbm, hbm, vmem],
        out_specs=vmem,
        scratch_shapes=[
            pltpu.VMEM((SKV_LOC, H, DH), jnp.float32),
            pltpu.VMEM((2, B, SKV_LOC, H * DH), jnp.bfloat16),
            pltpu.VMEM((N_DEV, 2, B, SKV_LOC, HD_LOC), jnp.bfloat16),
            pltpu.VMEM((B, SQ, DM), jnp.float32),
            pltpu.VMEM((N_DEV, B, RS_ROWS, DM), jnp.float32),
            pltpu.VMEM((B, RS_ROWS, DM), jnp.float32),
            pltpu.VMEM((N_DEV, B, RS_ROWS, DM), jnp.float32),
            pltpu.SemaphoreType.DMA,
            pltpu.SemaphoreType.DMA((N_DEV,)),
            pltpu.SemaphoreType.DMA((N_DEV,)),
            pltpu.SemaphoreType.DMA((N_DEV,)),
            pltpu.SemaphoreType.DMA((N_DEV,)),
            pltpu.SemaphoreType.DMA((N_DEV,)),
            pltpu.SemaphoreType.DMA((N_DEV,)),
        ],
        compiler_params=pltpu.CompilerParams(
            vmem_limit_bytes=100 * 1024 * 1024,
        ),
    )(x, Wq, K_ext, V_ext, Wo)


# device time: 179111 ns/iter; 1.2025x vs baseline; 1.2025x over previous
import jax
import jax.numpy as jnp
from jax import lax
from jax.experimental import pallas as pl
from jax.experimental.pallas import tpu as pltpu

N_DEV = 16
B = 2
SQ = 256
SKV_LOC = 256
SKV = SKV_LOC * N_DEV
H_LOC = 4
H = 64
DH = 64
DM = 512
BLK = 64
RS_ROWS = SQ // N_DEV
HD_LOC = H_LOC * DH


def kernel(x, Wq, K_ext, V_ext, Wo):
    kb16 = K_ext.astype(jnp.bfloat16).reshape(B, SKV_LOC, H * DH)
    vb16 = V_ext.astype(jnp.bfloat16).reshape(B, SKV_LOC, H * DH)

    def body(
        x_ref, wq_ref, k_ref, v_ref, wo_ref, out_ref,
        k_gath, v_gath, po_ref, rs_buf, red_ref, ag_buf,
        k_send, k_recv, v_send, v_recv,
        rs_send, rs_recv, ag_send, ag_recv,
    ):
        my = lax.axis_index("i")

        kv_sends = []
        for k in range(1, N_DEV):
            peer = lax.rem(my + k, N_DEV)
            for src_ref, gath, ssem, rsem in (
                (k_ref, k_gath, k_send, k_recv),
                (v_ref, v_gath, v_send, v_recv),
            ):
                rdma = pltpu.make_async_remote_copy(
                    src_ref=src_ref.at[:, :, pl.ds(HD_LOC * peer, HD_LOC)],
                    dst_ref=gath.at[my],
                    send_sem=ssem.at[peer],
                    recv_sem=rsem.at[my],
                    device_id=(peer,),
                    device_id_type=pl.DeviceIdType.MESH,
                )
                rdma.start()
                kv_sends.append(rdma)

        k_gath[pl.ds(my, 1)] = k_ref[:, :, pl.ds(HD_LOC * my, HD_LOC)][None]
        v_gath[pl.ds(my, 1)] = v_ref[:, :, pl.ds(HD_LOC * my, HD_LOC)][None]

        xb = x_ref[...].astype(jnp.bfloat16).reshape(B * SQ, DM)
        wqb = wq_ref[...].astype(jnp.bfloat16)
        q = jax.lax.dot_general(
            xb, wqb, (((1,), (0,)), ((), ())),
            preferred_element_type=jnp.float32,
        )

        qb = lax.broadcasted_iota(jnp.int32, (SQ, SKV), 0) // BLK
        kb = lax.broadcasted_iota(jnp.int32, (SQ, SKV), 1) // BLK
        mask = (qb == kb) | (kb == 0) | (lax.rem(qb + kb, 3) == 0)
        wob = wo_ref[...].astype(jnp.bfloat16)

        for k in range(1, N_DEV):
            src = lax.rem(my - k + N_DEV, N_DEV)
            for src_ref, gath, ssem, rsem in (
                (k_ref, k_gath, k_send, k_recv),
                (v_ref, v_gath, v_send, v_recv),
            ):
                pltpu.make_async_remote_copy(
                    src_ref=src_ref.at[:, :, pl.ds(0, HD_LOC)],
                    dst_ref=gath.at[src],
                    send_sem=ssem.at[src],
                    recv_sem=rsem.at[src],
                    device_id=(src,),
                    device_id_type=pl.DeviceIdType.MESH,
                ).wait_recv()
        for rdma in kv_sends:
            rdma.wait_send()

        po = []
        for b in range(B):
            acc = jnp.zeros((SQ, DM), jnp.float32)
            for h in range(H_LOC):
                kf = k_gath[:, b, :, pl.ds(DH * h, DH)].reshape(SKV, DH)
                vf = v_gath[:, b, :, pl.ds(DH * h, DH)].reshape(SKV, DH)
                qh = q[
                    b * SQ:(b + 1) * SQ, DH * h:DH * (h + 1)
                ].astype(jnp.bfloat16)
                s = jax.lax.dot_general(
                    qh, kf, (((1,), (1,)), ((), ())),
                    preferred_element_type=jnp.float32,
                )
                s = jnp.where(mask, s * 0.125, -1e9)
                m = jnp.max(s, axis=-1, keepdims=True)
                w = jnp.exp(s - m)
                w = w / jnp.sum(w, axis=-1, keepdims=True)
                ctx = jax.lax.dot_general(
                    w.astype(jnp.bfloat16), vf, (((1,), (0,)), ((), ())),
                    preferred_element_type=jnp.float32,
                )
                acc = acc + jax.lax.dot_general(
                    ctx.astype(jnp.bfloat16),
                    wob[h * DH:(h + 1) * DH, :],
                    (((1,), (0,)), ((), ())),
                    preferred_element_type=jnp.float32,
                )
            po.append(acc)
        po_ref[...] = jnp.stack(po)

        rs_sends = []
        for k in range(1, N_DEV):
            peer = lax.rem(my + k, N_DEV)
            rdma = pltpu.make_async_remote_copy(
                src_ref=po_ref.at[:, pl.ds(RS_ROWS * peer, RS_ROWS), :],
                dst_ref=rs_buf.at[my],
                send_sem=rs_send.at[peer],
                recv_sem=rs_recv.at[my],
                device_id=(peer,),
                device_id_type=pl.DeviceIdType.MESH,
            )
            rdma.start()
            rs_sends.append(rdma)
        own_chunk = po_ref[:, pl.ds(RS_ROWS * my, RS_ROWS), :]
        rs_buf[pl.ds(my, 1)] = own_chunk[None]
        for k in range(1, N_DEV):
            src = lax.rem(my - k + N_DEV, N_DEV)
            pltpu.make_async_remote_copy(
                src_ref=po_ref.at[:, pl.ds(0, RS_ROWS), :],
                dst_ref=rs_buf.at[src],
                send_sem=rs_send.at[src],
                recv_sem=rs_recv.at[src],
                device_id=(src,),
                device_id_type=pl.DeviceIdType.MESH,
            ).wait_recv()
        for rdma in rs_sends:
            rdma.wait_send()

        red = jnp.sum(rs_buf[...], axis=0)
        red_ref[...] = red

        ag_sends = []
        for k in range(1, N_DEV):
            peer = lax.rem(my + k, N_DEV)
            rdma = pltpu.make_async_remote_copy(
                src_ref=red_ref,
                dst_ref=ag_buf.at[my],
                send_sem=ag_send.at[peer],
                recv_sem=ag_recv.at[my],
                device_id=(peer,),
                device_id_type=pl.DeviceIdType.MESH,
            )
            rdma.start()
            ag_sends.append(rdma)
        ag_buf[pl.ds(my, 1)] = red[None]
        for k in range(1, N_DEV):
            src = lax.rem(my - k + N_DEV, N_DEV)
            pltpu.make_async_remote_copy(
                src_ref=red_ref,
                dst_ref=ag_buf.at[src],
                send_sem=ag_send.at[src],
                recv_sem=ag_recv.at[src],
                device_id=(src,),
                device_id_type=pl.DeviceIdType.MESH,
            ).wait_recv()
        for rdma in ag_sends:
            rdma.wait_send()

        out_ref[...] = jnp.transpose(
            ag_buf[...], (1, 0, 2, 3)
        ).reshape(B, SQ, DM)

    vmem = pl.BlockSpec(memory_space=pltpu.VMEM)
    return pl.pallas_call(
        body,
        out_shape=jax.ShapeDtypeStruct((B, SQ, DM), jnp.float32),
        in_specs=[vmem] * 5,
        out_specs=vmem,
        scratch_shapes=[
            pltpu.VMEM((N_DEV, B, SKV_LOC, HD_LOC), jnp.bfloat16),
            pltpu.VMEM((N_DEV, B, SKV_LOC, HD_LOC), jnp.bfloat16),
            pltpu.VMEM((B, SQ, DM), jnp.float32),
            pltpu.VMEM((N_DEV, B, RS_ROWS, DM), jnp.float32),
            pltpu.VMEM((B, RS_ROWS, DM), jnp.float32),
            pltpu.VMEM((N_DEV, B, RS_ROWS, DM), jnp.float32),
            pltpu.SemaphoreType.DMA((N_DEV,)),
            pltpu.SemaphoreType.DMA((N_DEV,)),
            pltpu.SemaphoreType.DMA((N_DEV,)),
            pltpu.SemaphoreType.DMA((N_DEV,)),
            pltpu.SemaphoreType.DMA((N_DEV,)),
            pltpu.SemaphoreType.DMA((N_DEV,)),
            pltpu.SemaphoreType.DMA((N_DEV,)),
            pltpu.SemaphoreType.DMA((N_DEV,)),
        ],
        compiler_params=pltpu.CompilerParams(
            vmem_limit_bytes=100 * 1024 * 1024,
        ),
    )(x, Wq, kb16, vb16, Wo)
